# baseline (speedup 1.0000x reference)
import functools

import jax
import jax.numpy as jnp
from jax import lax
from jax.experimental import pallas as pl
from jax.experimental.pallas import tpu as pltpu
from jax.experimental.pallas import tpu_sc as plsc

_NUM_CORES = 2
_NUM_SUBCORES = 16
_NUM_WORKERS = _NUM_CORES * _NUM_SUBCORES


def _gather_cls(label, cls_ctx):
    b = label.shape[0]
    n_ctx, d = cls_ctx.shape[1], cls_ctx.shape[2]
    b_per_w = b // _NUM_WORKERS

    mesh = plsc.VectorSubcoreMesh(core_axis_name="c", subcore_axis_name="s")

    @functools.partial(
        pl.kernel,
        mesh=mesh,
        out_type=jax.ShapeDtypeStruct((b, n_ctx, d), jnp.float32),
        scratch_types=[
            pltpu.VMEM((b_per_w,), jnp.int32),
            pltpu.VMEM((b_per_w, n_ctx, d), jnp.float32),
            pltpu.SemaphoreType.DMA,
        ],
    )
    def sc_gather(label_hbm, table_hbm, out_hbm, idx_v, rows_v, sem):
        wid = lax.axis_index("s") * _NUM_CORES + lax.axis_index("c")
        base = wid * b_per_w
        pltpu.sync_copy(label_hbm.at[pl.ds(base, b_per_w)], idx_v)
        pltpu.async_copy(table_hbm.at[idx_v], rows_v, sem).wait()
        pltpu.sync_copy(rows_v, out_hbm.at[pl.ds(base, b_per_w)])

    return sc_gather(label, cls_ctx)


def _assemble(cls, token_prefix, token_suffix):
    b, n_ctx, d = cls.shape
    pre = token_prefix.shape[1]
    suf = token_suffix.shape[1]
    tok = pre + n_ctx + suf
    b_blk = 8

    def body(pre_ref, cls_ref, suf_ref, out_ref):
        out_ref[:, 0:pre, :] = jnp.broadcast_to(pre_ref[:], (b_blk, pre, d))
        out_ref[:, pre:pre + n_ctx, :] = cls_ref[:]
        out_ref[:, pre + n_ctx:tok, :] = jnp.broadcast_to(
            suf_ref[:], (b_blk, suf, d))

    return pl.pallas_call(
        body,
        grid=(b // b_blk,),
        in_specs=[
            pl.BlockSpec((1, pre, d), lambda i: (0, 0, 0)),
            pl.BlockSpec((b_blk, n_ctx, d), lambda i: (i, 0, 0)),
            pl.BlockSpec((1, suf, d), lambda i: (0, 0, 0)),
        ],
        out_specs=pl.BlockSpec((b_blk, tok, d), lambda i: (i, 0, 0)),
        out_shape=jax.ShapeDtypeStruct((b, tok, d), jnp.float32),
    )(token_prefix, cls, token_suffix)


def kernel(label, cls_ctx, token_prefix, token_suffix):
    cls = _gather_cls(label, cls_ctx)
    return _assemble(cls, token_prefix, token_suffix)
